# baseline (speedup 1.0000x reference)
import functools
import math

import numpy as np
import jax
import jax.numpy as jnp
from jax import lax
from jax.experimental import pallas as pl
from jax.experimental.pallas import tpu as pltpu
from jax.experimental.pallas import tpu_sc as plsc

VOCAB = 100000
EMB = 64
CELL_IN = 128
BATCH = 4096
L = 200


def _make_pe_np():
    position = np.arange(0, L, dtype=np.float32)[:, None]
    div_term = np.exp(
        np.arange(0, EMB, 2, dtype=np.float32) * (-math.log(10000.0) / EMB))
    pe = np.zeros((L, EMB), dtype=np.float32)
    pe[:, 0::2] = np.sin(position * div_term)
    pe[:, 1::2] = np.cos(position * div_term)
    return pe


_PE = _make_pe_np()


def _cell_proj_tc(cell_emb, W, b):
    BB = 512

    def body(x_ref, w_ref, b_ref, o_ref):
        o_ref[...] = (
            jnp.dot(x_ref[...], w_ref[...].T, preferred_element_type=jnp.float32)
            + b_ref[...])

    return pl.pallas_call(
        body,
        grid=(BATCH // BB,),
        in_specs=[
            pl.BlockSpec((BB, CELL_IN), lambda i: (i, 0)),
            pl.BlockSpec((EMB, CELL_IN), lambda i: (0, 0)),
            pl.BlockSpec((1, EMB), lambda i: (0, 0)),
        ],
        out_specs=pl.BlockSpec((BB, EMB), lambda i: (i, 0)),
        out_shape=jax.ShapeDtypeStruct((BATCH, EMB), jnp.float32),
    )(cell_emb, W, b.reshape(1, EMB))


def _sc_embed(seqs_flat, emb_table, cell_flat, gamma, beta, pe_flat):
    info = plsc.get_sparse_core_info()
    NC, NS = info.num_cores, info.num_subcores
    NW = NC * NS
    BPW = BATCH // NW

    mesh = plsc.VectorSubcoreMesh(core_axis_name="c", subcore_axis_name="s")

    @functools.partial(
        pl.kernel,
        mesh=mesh,
        compiler_params=pltpu.CompilerParams(
            needs_layout_passes=False, use_tc_tiling_on_sc=False),
        out_type=jax.ShapeDtypeStruct((L, EMB // 8, NW, 8, BPW), jnp.float32),
        scratch_types=[
            pltpu.VMEM((BPW * L,), jnp.int32),
            pltpu.VMEM((L, BPW), jnp.int32),
            pltpu.VMEM((BPW * EMB,), jnp.float32),
            pltpu.VMEM((EMB, BPW), jnp.float32),
            pltpu.VMEM((L * EMB,), jnp.float32),
            pltpu.VMEM((EMB,), jnp.float32),
            pltpu.VMEM((EMB,), jnp.float32),
            pltpu.VMEM((2, BPW, EMB), jnp.float32),
            pltpu.VMEM((2, EMB // 8, 8, BPW), jnp.float32),
            [pltpu.SemaphoreType.DMA] * 2,
            [pltpu.SemaphoreType.DMA] * 2,
        ],
    )
    def k(seqs_hbm, table_hbm, cell_hbm, gamma_hbm, beta_hbm, pe_hbm,
          out_hbm, seqs_v, seqsT_v, cell_v, cellT_v, pe_v, g_v, be_v,
          rows_v, brick_v, gsem, osem):
        wid = lax.axis_index("s") * NC + lax.axis_index("c")
        b0 = wid * BPW
        pltpu.sync_copy(seqs_hbm.at[pl.ds(b0 * L, BPW * L)], seqs_v)
        pltpu.sync_copy(cell_hbm.at[pl.ds(b0 * EMB, BPW * EMB)], cell_v)
        pltpu.sync_copy(pe_hbm.at[pl.ds(0, L * EMB)], pe_v)
        pltpu.sync_copy(gamma_hbm, g_v)
        pltpu.sync_copy(beta_hbm, be_v)

        lanes = lax.iota(jnp.int32, 16)
        biota = [lanes + 16 * bg for bg in range(BPW // 16)]
        biotaL = [b * L for b in biota]
        biotaE = [b * EMB for b in biota]

        def transpose_seqs(p, _):
            for bg in range(BPW // 16):
                v = plsc.load_gather(seqs_v, [biotaL[bg] + p])
                seqsT_v[p, pl.ds(16 * bg, 16)] = v
            return 0

        lax.fori_loop(0, L, transpose_seqs, 0)

        def transpose_cell(e, _):
            for bg in range(BPW // 16):
                v = plsc.load_gather(cell_v, [biotaE[bg] + e])
                cellT_v[e, pl.ds(16 * bg, 16)] = v
            return 0

        lax.fori_loop(0, EMB, transpose_cell, 0)

        NBG = BPW // 16

        def gather_copy(p, s):
            return pltpu.make_async_copy(
                table_hbm.at[seqsT_v.at[p]], rows_v.at[s], gsem[s])

        def out_copy(p, s):
            return pltpu.make_async_copy(
                brick_v.at[s], out_hbm.at[p, :, wid], osem[s])

        def compute(p, s):
            rows2d = rows_v.at[s]

            def pass1(c16, carry):
                accs = list(carry[:NBG])
                acc2s = list(carry[NBG:])
                peC = pe_v[pl.ds(p * EMB + c16 * 16, 16)]
                for j in range(16):
                    e = c16 * 16 + j
                    pe_s = peC[j]
                    eh = c16 * 2 + j // 8
                    el = j % 8
                    for bg in range(NBG):
                        g = plsc.load_gather(
                            rows2d, [biota[bg], jnp.full((16,), e, jnp.int32)])
                        t = g + (cellT_v[e, pl.ds(16 * bg, 16)] + pe_s)
                        brick_v[s, eh, el, pl.ds(16 * bg, 16)] = t
                        accs[bg] = accs[bg] + t
                        acc2s[bg] = acc2s[bg] + t * t
                return tuple(accs) + tuple(acc2s)

            zero = jnp.zeros((16,), jnp.float32)
            carry = lax.fori_loop(0, EMB // 16, pass1, (zero,) * (2 * NBG))

            invs = []
            mivs = []
            for bg in range(NBG):
                mu = carry[bg] * (1.0 / EMB)
                ex2 = carry[NBG + bg] * (1.0 / EMB)
                v = (ex2 - mu * mu) + 1e-12
                i32 = plsc.bitcast(v, jnp.int32)
                i32 = jnp.int32(0x5F3759DF) - lax.shift_right_logical(i32, 1)
                y = plsc.bitcast(i32, jnp.float32)
                h = 0.5 * v
                y = y * (1.5 - h * y * y)
                y = y * (1.5 - h * y * y)
                y = y * (1.5 - h * y * y)
                invs.append(y)
                mivs.append(mu * y)

            def pass2(c16, _):
                gC = g_v[pl.ds(c16 * 16, 16)]
                bC = be_v[pl.ds(c16 * 16, 16)]
                for j in range(16):
                    g_s = gC[j]
                    b_s = bC[j]
                    eh = c16 * 2 + j // 8
                    el = j % 8
                    for bg in range(NBG):
                        t = brick_v[s, eh, el, pl.ds(16 * bg, 16)]
                        a = invs[bg] * g_s
                        c = b_s - mivs[bg] * g_s
                        brick_v[s, eh, el, pl.ds(16 * bg, 16)] = t * a + c
                return 0

            lax.fori_loop(0, EMB // 16, pass2, 0)

        gather_copy(0, 0).start()

        def pair_body(gidx, _):
            for s in range(2):
                p = 2 * gidx + s
                ns = 1 - s

                @pl.when(p + 1 < L)
                def _():
                    gather_copy(p + 1, ns).start()

                @pl.when(p >= 2)
                def _():
                    out_copy(p - 2, s).wait()

                gather_copy(p, s).wait()
                compute(p, s)
                out_copy(p, s).start()
            return 0

        lax.fori_loop(0, L // 2, pair_body, 0)
        out_copy(L - 2, 0).wait()
        out_copy(L - 1, 1).wait()

    return k(seqs_flat, emb_table, cell_flat, gamma, beta, pe_flat)


def kernel(seqs, cell_emb, emb_table, W, b, gamma, beta):
    cell_proj = _cell_proj_tc(cell_emb, W, b)
    pe = jnp.asarray(_PE.reshape(-1))
    out5 = _sc_embed(seqs.astype(jnp.int32).reshape(-1), emb_table,
                     cell_proj.reshape(-1), gamma, beta, pe)
    out = out5.transpose(2, 4, 0, 1, 3).reshape(BATCH, L, EMB)
    return (out, cell_proj)


# speedup vs baseline: 7.3391x; 7.3391x over previous
import functools
import math

import numpy as np
import jax
import jax.numpy as jnp
from jax import lax
from jax.experimental import pallas as pl
from jax.experimental.pallas import tpu as pltpu
from jax.experimental.pallas import tpu_sc as plsc

VOCAB = 100000
EMB = 64
CELL_IN = 128
BATCH = 4096
L = 200


def _make_pe_np():
    position = np.arange(0, L, dtype=np.float32)[:, None]
    div_term = np.exp(
        np.arange(0, EMB, 2, dtype=np.float32) * (-math.log(10000.0) / EMB))
    pe = np.zeros((L, EMB), dtype=np.float32)
    pe[:, 0::2] = np.sin(position * div_term)
    pe[:, 1::2] = np.cos(position * div_term)
    return pe


_PE = _make_pe_np()


def _cell_proj_tc(cell_emb, W, b):
    BB = 512

    def body(x_ref, w_ref, b_ref, o_ref):
        o_ref[...] = (
            jnp.dot(x_ref[...], w_ref[...].T, preferred_element_type=jnp.float32)
            + b_ref[...])

    return pl.pallas_call(
        body,
        grid=(BATCH // BB,),
        in_specs=[
            pl.BlockSpec((BB, CELL_IN), lambda i: (i, 0)),
            pl.BlockSpec((EMB, CELL_IN), lambda i: (0, 0)),
            pl.BlockSpec((1, EMB), lambda i: (0, 0)),
        ],
        out_specs=pl.BlockSpec((BB, EMB), lambda i: (i, 0)),
        out_shape=jax.ShapeDtypeStruct((BATCH, EMB), jnp.float32),
    )(cell_emb, W, b.reshape(1, EMB))


def _sc_embed(seqs_flat, emb_table, cell_flat, gamma, beta, pe_flat):
    info = plsc.get_sparse_core_info()
    NC, NS = info.num_cores, info.num_subcores
    NW = NC * NS
    BPW = BATCH // NW

    mesh = plsc.VectorSubcoreMesh(core_axis_name="c", subcore_axis_name="s")

    @functools.partial(
        pl.kernel,
        mesh=mesh,
        compiler_params=pltpu.CompilerParams(
            needs_layout_passes=False, use_tc_tiling_on_sc=False),
        out_type=jax.ShapeDtypeStruct((L, EMB // 8, NW, 8, BPW), jnp.float32),
        scratch_types=[
            pltpu.VMEM((BPW * L,), jnp.int32),
            pltpu.VMEM((L, BPW), jnp.int32),
            pltpu.VMEM((BPW * EMB,), jnp.float32),
            pltpu.VMEM((L * EMB,), jnp.float32),
            pltpu.VMEM((EMB,), jnp.float32),
            pltpu.VMEM((EMB,), jnp.float32),
            pltpu.VMEM((2, BPW, EMB), jnp.float32),
            pltpu.VMEM((2, EMB // 8, 8, BPW), jnp.float32),
            [pltpu.SemaphoreType.DMA] * 2,
            [pltpu.SemaphoreType.DMA] * 2,
        ],
    )
    def k(seqs_hbm, table_hbm, cell_hbm, gamma_hbm, beta_hbm, pe_hbm,
          out_hbm, seqs_v, seqsT_v, cell_v, pe_v, g_v, be_v,
          rows_v, brick_v, gsem, osem):
        wid = lax.axis_index("s") * NC + lax.axis_index("c")
        b0 = wid * BPW
        pltpu.sync_copy(seqs_hbm.at[pl.ds(b0 * L, BPW * L)], seqs_v)
        pltpu.sync_copy(cell_hbm.at[pl.ds(b0 * EMB, BPW * EMB)], cell_v)
        pltpu.sync_copy(pe_hbm.at[pl.ds(0, L * EMB)], pe_v)
        pltpu.sync_copy(gamma_hbm, g_v)
        pltpu.sync_copy(beta_hbm, be_v)

        lanes = lax.iota(jnp.int32, 16)
        biota = [lanes + 16 * bg for bg in range(BPW // 16)]
        biotaL = [b * L for b in biota]
        biotaE = [b * EMB for b in biota]

        def transpose_seqs(p, _):
            for bg in range(BPW // 16):
                v = plsc.load_gather(seqs_v, [biotaL[bg] + p])
                seqsT_v[p, pl.ds(16 * bg, 16)] = v
            return 0

        lax.fori_loop(0, L, transpose_seqs, 0)

        gj = [g_v[pl.ds(16 * j, 16)] for j in range(4)]
        bj = [be_v[pl.ds(16 * j, 16)] for j in range(4)]
        ehv = [lax.shift_right_logical(lanes, 3) + 2 * j for j in range(4)]
        elv = lax.bitwise_and(lanes, 7)

        def gather_copy(p, s):
            return pltpu.make_async_copy(
                table_hbm.at[seqsT_v.at[p]], rows_v.at[s], gsem[s])

        def out_copy(p, s):
            return pltpu.make_async_copy(
                brick_v.at[s], out_hbm.at[p, :, wid], osem[s])

        def compute(p, s):
            rows2d = rows_v.at[s]
            brick3d = brick_v.at[s]
            pe_j = [pe_v[pl.ds(p * EMB + 16 * j, 16)] for j in range(4)]

            @functools.partial(plsc.parallel_loop, 0, BPW, unroll=4)
            def rowbody(b_l):
                t = [rows2d[b_l, pl.ds(16 * j, 16)]
                     + (cell_v[pl.ds(b_l * EMB + 16 * j, 16)] + pe_j[j])
                     for j in range(4)]
                ssum = (t[0] + t[1]) + (t[2] + t[3])
                q = (t[0] * t[0] + t[1] * t[1]) + (t[2] * t[2] + t[3] * t[3])
                mu = jnp.sum(ssum) * (1.0 / EMB)
                ex2 = jnp.sum(q) * (1.0 / EMB)
                v = (ex2 - mu * mu) + 1e-12
                i32 = lax.bitcast_convert_type(v, jnp.int32)
                i32 = jnp.int32(0x5F3759DF) - lax.shift_right_logical(i32, 1)
                y = lax.bitcast_convert_type(i32, jnp.float32)
                h = 0.5 * v
                y = y * (1.5 - h * y * y)
                y = y * (1.5 - h * y * y)
                y = y * (1.5 - h * y * y)
                blv = jnp.full((16,), b_l, jnp.int32)
                for j in range(4):
                    o = (t[j] - mu) * (gj[j] * y) + bj[j]
                    plsc.store_scatter(brick3d, [ehv[j], elv, blv], o)

        gather_copy(0, 0).start()

        def pair_body(gidx, _):
            for s in range(2):
                p = 2 * gidx + s
                ns = 1 - s

                @pl.when(p + 1 < L)
                def _():
                    gather_copy(p + 1, ns).start()

                @pl.when(p >= 2)
                def _():
                    out_copy(p - 2, s).wait()

                gather_copy(p, s).wait()
                compute(p, s)
                out_copy(p, s).start()
            return 0

        lax.fori_loop(0, L // 2, pair_body, 0)
        out_copy(L - 2, 0).wait()
        out_copy(L - 1, 1).wait()

    return k(seqs_flat, emb_table, cell_flat, gamma, beta, pe_flat)


def kernel(seqs, cell_emb, emb_table, W, b, gamma, beta):
    cell_proj = _cell_proj_tc(cell_emb, W, b)
    pe = jnp.asarray(_PE.reshape(-1))
    out5 = _sc_embed(seqs.astype(jnp.int32).reshape(-1), emb_table,
                     cell_proj.reshape(-1), gamma, beta, pe)
    out = out5.transpose(2, 4, 0, 1, 3).reshape(BATCH, L, EMB)
    return (out, cell_proj)
